# baseline (speedup 1.0000x reference)
import functools

import jax
import jax.numpy as jnp
from jax import lax
from jax.experimental import pallas as pl
from jax.experimental.pallas import tpu as pltpu
from jax.experimental.pallas import tpu_sc as plsc

B = 16384
H = 200
D = 64
NC = 2
NS = 16
NW = NC * NS
RW = B // NW
CH = 128
NCH = RW // CH


CH2 = H - CH


def _sc_body(
    x_hbm, emb_hbm, out_hbm,
    idx_a, idx_b, rows_a, rows_b, acc_v, sem_a, sem_b, isem,
):
    c = lax.axis_index("c")
    s = lax.axis_index("s")
    wid = c * NS + s
    base = wid * RW

    def fire_idx(b, idx_ref):
        pltpu.async_copy(x_hbm.at[base + b], idx_ref, isem)

    def wait_idx(idx_ref):
        pltpu.make_async_copy(x_hbm.at[0], idx_ref, isem).wait()

    def fire_gathers(idx_ref, rows_ref, sem):
        pltpu.async_copy(
            emb_hbm.at[idx_ref.at[pl.ds(0, CH)]], rows_ref.at[pl.ds(0, CH)], sem
        )
        pltpu.async_copy(
            emb_hbm.at[idx_ref.at[pl.ds(CH, CH2)]],
            rows_ref.at[pl.ds(CH, CH2)],
            sem,
        )

    def wait_gathers(rows_ref, sem):
        pltpu.make_async_copy(emb_hbm.at[pl.ds(0, H)], rows_ref, sem).wait()

    z = jnp.zeros((16,), jnp.float32)

    def reduce_into(rows_ref, b):
        @plsc.parallel_loop(0, H // 2, unroll=4, carry=(z,) * 8)
        def _red(r, p):
            lo = [rows_ref[2 * r, pl.ds(k * 16, 16)] for k in range(4)]
            hi = [rows_ref[2 * r + 1, pl.ds(k * 16, 16)] for k in range(4)]
            return tuple(p[k] + lo[k] for k in range(4)) + tuple(
                p[4 + k] + hi[k] for k in range(4)
            )

        for k in range(4):
            acc_v[b, pl.ds(k * 16, 16)] = _red[k] + _red[4 + k]

    pltpu.sync_copy(x_hbm.at[base], idx_a)
    fire_gathers(idx_a, rows_a, sem_a)
    fire_idx(1, idx_b)

    def pair(i, carry):
        b = 2 * i
        wait_idx(idx_b)
        fire_gathers(idx_b, rows_b, sem_b)
        wait_gathers(rows_a, sem_a)
        fire_idx(b + 2, idx_a)
        reduce_into(rows_a, b)

        wait_idx(idx_a)
        fire_gathers(idx_a, rows_a, sem_a)
        wait_gathers(rows_b, sem_b)
        fire_idx(b + 3, idx_b)
        reduce_into(rows_b, b + 1)
        return carry

    lax.fori_loop(0, (RW - 2) // 2, pair, 0)

    wait_idx(idx_b)
    fire_gathers(idx_b, rows_b, sem_b)
    wait_gathers(rows_a, sem_a)
    reduce_into(rows_a, RW - 2)
    wait_gathers(rows_b, sem_b)
    reduce_into(rows_b, RW - 1)

    pltpu.sync_copy(acc_v, out_hbm.at[pl.ds(base, RW)])


@jax.jit
def _sc_sum(x, emb):
    mesh = plsc.VectorSubcoreMesh(core_axis_name="c", subcore_axis_name="s")
    fn = pl.kernel(
        _sc_body,
        out_type=jax.ShapeDtypeStruct((B, D), jnp.float32),
        mesh=mesh,
        scratch_types=[
            pltpu.VMEM((H,), jnp.int32),
            pltpu.VMEM((H,), jnp.int32),
            pltpu.VMEM((H, D), jnp.float32),
            pltpu.VMEM((H, D), jnp.float32),
            pltpu.VMEM((RW, D), jnp.float32),
            pltpu.SemaphoreType.DMA,
            pltpu.SemaphoreType.DMA,
            pltpu.SemaphoreType.DMA,
        ],
        compiler_params=pltpu.CompilerParams(use_tc_tiling_on_sc=False),
    )
    return fn(x, emb)


BLK = 512


def _tc_body(x_ref, sum_ref, w_ref, b_ref, o_ref):
    cnt = jnp.sum((x_ref[...] != 0).astype(jnp.float32), axis=1, keepdims=True)
    mean = sum_ref[...] / (cnt + 1e-6)
    o_ref[...] = (
        lax.dot_general(
            mean, w_ref[...], (((1,), (1,)), ((), ())),
            preferred_element_type=jnp.float32,
        )
        + b_ref[...]
    )


@jax.jit
def _tc_finish(x, summed, W, b2):
    return pl.pallas_call(
        _tc_body,
        grid=(B // BLK,),
        in_specs=[
            pl.BlockSpec((BLK, H), lambda i: (i, 0)),
            pl.BlockSpec((BLK, D), lambda i: (i, 0)),
            pl.BlockSpec((D, D), lambda i: (0, 0)),
            pl.BlockSpec((1, D), lambda i: (0, 0)),
        ],
        out_specs=pl.BlockSpec((BLK, D), lambda i: (i, 0)),
        out_shape=jax.ShapeDtypeStruct((B, D), jnp.float32),
    )(x, summed, W, b2)


def kernel(x, lengths, emb, W, b):
    x = jnp.asarray(x, jnp.int32)
    summed = _sc_sum(x, emb)
    return _tc_finish(x, summed, W, b.reshape(1, D))


# speedup vs baseline: 1.1451x; 1.1451x over previous
import functools

import jax
import jax.numpy as jnp
from jax import lax
from jax.experimental import pallas as pl
from jax.experimental.pallas import tpu as pltpu
from jax.experimental.pallas import tpu_sc as plsc

B = 16384
H = 200
D = 64
NC = 2
NS = 16
NW = NC * NS
RW = B // NW
CH = 128
NCH = RW // CH


CH2 = H - CH


NSLOT = 4


def _sc_body(
    x_hbm, emb_hbm, out_hbm,
    idx_a, idx_b, idx_c, idx_d,
    rows_a, rows_b, rows_c, rows_d,
    acc_v,
    sem_a, sem_b, sem_c, sem_d,
    isem_a, isem_b, isem_c, isem_d,
):
    idx = [idx_a, idx_b, idx_c, idx_d]
    rows = [rows_a, rows_b, rows_c, rows_d]
    gsem = [sem_a, sem_b, sem_c, sem_d]
    isem = [isem_a, isem_b, isem_c, isem_d]

    c = lax.axis_index("c")
    s = lax.axis_index("s")
    wid = c * NS + s
    base = wid * RW

    def fire_idx(b, j):
        pltpu.async_copy(x_hbm.at[base + b], idx[j], isem[j])

    def wait_idx(j):
        pltpu.make_async_copy(x_hbm.at[0], idx[j], isem[j]).wait()

    def fire_gathers(idx_ref, rows_ref, sem):
        pltpu.async_copy(
            emb_hbm.at[idx_ref.at[pl.ds(0, CH)]], rows_ref.at[pl.ds(0, CH)], sem
        )
        pltpu.async_copy(
            emb_hbm.at[idx_ref.at[pl.ds(CH, CH2)]],
            rows_ref.at[pl.ds(CH, CH2)],
            sem,
        )

    def wait_gathers(rows_ref, sem):
        pltpu.make_async_copy(emb_hbm.at[pl.ds(0, H)], rows_ref, sem).wait()

    z = jnp.zeros((16,), jnp.float32)

    def reduce_into(rows_ref, b):
        @plsc.parallel_loop(0, H // 2, unroll=4, carry=(z,) * 8)
        def _red(r, p):
            lo = [rows_ref[2 * r, pl.ds(k * 16, 16)] for k in range(4)]
            hi = [rows_ref[2 * r + 1, pl.ds(k * 16, 16)] for k in range(4)]
            return tuple(p[k] + lo[k] for k in range(4)) + tuple(
                p[4 + k] + hi[k] for k in range(4)
            )

        for k in range(4):
            acc_v[b, pl.ds(k * 16, 16)] = _red[k] + _red[4 + k]

    for j in range(NSLOT):
        fire_idx(j, j)
    for j in range(2):
        wait_idx(j)
        fire_gathers(idx[j], rows[j], gsem[j])

    def quad(i, carry):
        b4 = 4 * i
        for j in range(NSLOT):
            jg = (j + 2) % NSLOT
            wait_idx(jg)
            fire_gathers(idx[jg], rows[jg], gsem[jg])
            wait_gathers(rows[j], gsem[j])
            fire_idx(b4 + j + 4, j)
            reduce_into(rows[j], b4 + j)
        return carry

    lax.fori_loop(0, (RW - 4) // 4, quad, 0)

    for j in range(2):
        wait_idx(j + 2)
        fire_gathers(idx[j + 2], rows[j + 2], gsem[j + 2])
        wait_gathers(rows[j], gsem[j])
        reduce_into(rows[j], RW - 4 + j)
    for j in range(2, 4):
        wait_gathers(rows[j], gsem[j])
        reduce_into(rows[j], RW - 4 + j)

    pltpu.sync_copy(acc_v, out_hbm.at[pl.ds(base, RW)])


@jax.jit
def _sc_sum(x, emb):
    mesh = plsc.VectorSubcoreMesh(core_axis_name="c", subcore_axis_name="s")
    fn = pl.kernel(
        _sc_body,
        out_type=jax.ShapeDtypeStruct((B, D), jnp.float32),
        mesh=mesh,
        scratch_types=(
            [pltpu.VMEM((H,), jnp.int32)] * 4
            + [pltpu.VMEM((H, D), jnp.float32)] * 4
            + [pltpu.VMEM((RW, D), jnp.float32)]
            + [pltpu.SemaphoreType.DMA] * 8
        ),
        compiler_params=pltpu.CompilerParams(use_tc_tiling_on_sc=False),
    )
    return fn(x, emb)


BLK = 512


def _tc_body(x_ref, sum_ref, w_ref, b_ref, o_ref):
    cnt = jnp.sum((x_ref[...] != 0).astype(jnp.float32), axis=1, keepdims=True)
    mean = sum_ref[...] / (cnt + 1e-6)
    o_ref[...] = (
        lax.dot_general(
            mean, w_ref[...], (((1,), (1,)), ((), ())),
            preferred_element_type=jnp.float32,
        )
        + b_ref[...]
    )


@jax.jit
def _tc_finish(x, summed, W, b2):
    return pl.pallas_call(
        _tc_body,
        grid=(B // BLK,),
        in_specs=[
            pl.BlockSpec((BLK, H), lambda i: (i, 0)),
            pl.BlockSpec((BLK, D), lambda i: (i, 0)),
            pl.BlockSpec((D, D), lambda i: (0, 0)),
            pl.BlockSpec((1, D), lambda i: (0, 0)),
        ],
        out_specs=pl.BlockSpec((BLK, D), lambda i: (i, 0)),
        out_shape=jax.ShapeDtypeStruct((B, D), jnp.float32),
    )(x, summed, W, b2)


def kernel(x, lengths, emb, W, b):
    x = jnp.asarray(x, jnp.int32)
    summed = _sc_sum(x, emb)
    return _tc_finish(x, summed, W, b.reshape(1, D))
